# baseline (speedup 1.0000x reference)
import functools

import jax
import jax.numpy as jnp
from jax import lax
from jax.experimental import pallas as pl
from jax.experimental.pallas import tpu as pltpu

B, S, H = 2, 4096, 768
E, TOPK, FF = 16, 2, 384
SFF = 384 * 2
T = B * S
N = T * TOPK
TM = 256
TG = 256
NPAD = N + E * TG
NT = NPAD // TG


def _gate_shared_body(x_ref, gw_ref, s1_ref, s2_ref, s3_ref,
                      idx_ref, w_ref, sh_ref):
    x = x_ref[...]
    logits = lax.dot_general(x, gw_ref[...], (((1,), (1,)), ((), ())),
                             preferred_element_type=jnp.float32)
    m = jnp.max(logits, axis=-1, keepdims=True)
    p = jnp.exp(logits - m)
    s = p / jnp.sum(p, axis=-1, keepdims=True)
    iota = lax.broadcasted_iota(jnp.int32, (TM, E), 1)
    m1 = jnp.max(s, axis=-1, keepdims=True)
    i1 = jnp.min(jnp.where(s == m1, iota, E), axis=-1, keepdims=True)
    oh1 = iota == i1
    s2 = jnp.where(oh1, -1.0, s)
    m2 = jnp.max(s2, axis=-1, keepdims=True)
    i2 = jnp.min(jnp.where(s2 == m2, iota, E), axis=-1, keepdims=True)
    denom = m1 + m2 + 1e-20
    idx_ref[...] = jnp.concatenate([i1, i2], axis=1)
    w_ref[...] = jnp.concatenate([m1 / denom, m2 / denom], axis=1)
    xb = x.astype(jnp.bfloat16)
    g = lax.dot_general(xb, s1_ref[...], (((1,), (1,)), ((), ())),
                        preferred_element_type=jnp.float32)
    u = lax.dot_general(xb, s2_ref[...], (((1,), (1,)), ((), ())),
                        preferred_element_type=jnp.float32)
    a = ((g * jax.nn.sigmoid(g)) * u).astype(jnp.bfloat16)
    sh_ref[...] = lax.dot_general(a, s3_ref[...], (((1,), (0,)), ((), ())),
                                  preferred_element_type=jnp.float32)


def _gate_shared(x, gw, s1, s2, s3):
    return pl.pallas_call(
        _gate_shared_body,
        grid=(T // TM,),
        in_specs=[
            pl.BlockSpec((TM, H), lambda i: (i, 0)),
            pl.BlockSpec((E, H), lambda i: (0, 0)),
            pl.BlockSpec((SFF, H), lambda i: (0, 0)),
            pl.BlockSpec((SFF, H), lambda i: (0, 0)),
            pl.BlockSpec((SFF, H), lambda i: (0, 0)),
        ],
        out_specs=[
            pl.BlockSpec((TM, TOPK), lambda i: (i, 0)),
            pl.BlockSpec((TM, TOPK), lambda i: (i, 0)),
            pl.BlockSpec((TM, H), lambda i: (i, 0)),
        ],
        out_shape=[
            jax.ShapeDtypeStruct((T, TOPK), jnp.int32),
            jax.ShapeDtypeStruct((T, TOPK), jnp.float32),
            jax.ShapeDtypeStruct((T, H), jnp.float32),
        ],
    )(x, gw, s1, s2, s3)


def _grouped_mlp_body(eid_ref, xs_ref, w_ref, wg_ref, wu_ref, wd_ref,
                      ys_ref):
    xb = xs_ref[...].astype(jnp.bfloat16)
    g = lax.dot_general(xb, wg_ref[0], (((1,), (1,)), ((), ())),
                        preferred_element_type=jnp.float32)
    u = lax.dot_general(xb, wu_ref[0], (((1,), (1,)), ((), ())),
                        preferred_element_type=jnp.float32)
    a = ((g * jax.nn.sigmoid(g)) * u * w_ref[...]).astype(jnp.bfloat16)
    ys_ref[...] = lax.dot_general(a, wd_ref[0], (((1,), (1,)), ((), ())),
                                  preferred_element_type=jnp.float32)


def _grouped_mlp(tile_eid, xs, sorted_w, wg, wu, wd):
    grid_spec = pltpu.PrefetchScalarGridSpec(
        num_scalar_prefetch=1,
        grid=(NT,),
        in_specs=[
            pl.BlockSpec((TG, H), lambda i, eid: (i, 0)),
            pl.BlockSpec((TG, 1), lambda i, eid: (i, 0)),
            pl.BlockSpec((1, FF, H), lambda i, eid: (eid[i], 0, 0)),
            pl.BlockSpec((1, FF, H), lambda i, eid: (eid[i], 0, 0)),
            pl.BlockSpec((1, H, FF), lambda i, eid: (eid[i], 0, 0)),
        ],
        out_specs=pl.BlockSpec((TG, H), lambda i, eid: (i, 0)),
    )
    return pl.pallas_call(
        _grouped_mlp_body,
        grid_spec=grid_spec,
        out_shape=jax.ShapeDtypeStruct((NPAD, H), jnp.float32),
    )(tile_eid, xs, sorted_w, wg, wu, wd)


def _route(idx, wts):
    eid = idx.reshape(N)
    wf = wts.reshape(N)
    oh = (eid[:, None] == jnp.arange(E, dtype=jnp.int32)[None, :])
    ranks = jnp.cumsum(oh.astype(jnp.int32), axis=0)
    counts = ranks[-1]
    padded = ((counts + TG - 1) // TG) * TG
    pcum = jnp.cumsum(padded)
    base = pcum - padded
    rank_n = jnp.take_along_axis(ranks, eid[:, None], axis=1)[:, 0] - 1
    pos = base[eid] + rank_n
    sorted_tok = jnp.zeros((NPAD,), jnp.int32).at[pos].set(
        jnp.arange(N, dtype=jnp.int32) // TOPK)
    sorted_w = jnp.zeros((NPAD,), jnp.float32).at[pos].set(wf)
    tile_start = jnp.arange(NT, dtype=jnp.int32) * TG
    tile_eid = jnp.minimum(
        jnp.sum((tile_start[:, None] >= pcum[None, :]).astype(jnp.int32),
                axis=1), E - 1).astype(jnp.int32)
    return pos, sorted_tok, sorted_w, tile_eid


@jax.jit
def _moe(x, gw, wg, wu, wd, s1, s2, s3):
    idx, wts, shared = _gate_shared(x, gw, s1, s2, s3)
    pos, sorted_tok, sorted_w, tile_eid = _route(idx, wts)
    xs = jnp.take(x, sorted_tok, axis=0)
    ys = _grouped_mlp(tile_eid, xs, sorted_w[:, None], wg, wu, wd)
    pos2 = pos.reshape(T, TOPK)
    y = (shared + jnp.take(ys, pos2[:, 0], axis=0)
         + jnp.take(ys, pos2[:, 1], axis=0))
    return y


def kernel(hidden_states, gate_weight, Wg, Wu, Wd, sWg, sWu, sWd):
    x = hidden_states.reshape(T, H)
    wg = Wg.astype(jnp.bfloat16)
    wu = Wu.astype(jnp.bfloat16)
    wd = Wd.astype(jnp.bfloat16)
    s1 = sWg.astype(jnp.bfloat16)
    s2 = sWu.astype(jnp.bfloat16)
    s3 = sWd.T.astype(jnp.bfloat16)
    y = _moe(x, gate_weight, wg, wu, wd, s1, s2, s3)
    return y.reshape(B, S, H)
